# baseline (speedup 1.0000x reference)
import functools

import jax
import jax.numpy as jnp
import numpy as np
from jax import lax
from jax.experimental import pallas as pl
from jax.experimental.pallas import tpu as pltpu
from jax.experimental.pallas import tpu_sc as plsc

B, T, D = 2, 2048, 1024
E, K = 8, 2
FF = 4 * D
N = B * T
A = N * K
TM = 128
NT = A // TM + E - 1
A_PAD = NT * TM
DH = D // 2

_INV_SQRT2 = float(1.0 / np.sqrt(2.0))


def _pack_bf16(a, b):
    def rnd(v):
        bits = lax.bitcast_convert_type(v, jnp.int32)
        lsb = lax.shift_right_logical(bits, 16) & 1
        return lax.shift_right_logical(bits + 0x7FFF + lsb, 16)
    return rnd(a) | lax.shift_left(rnd(b), 16)


def _unpack_bf16(w):
    a = lax.bitcast_convert_type(lax.shift_left(w, 16), jnp.float32)
    b = lax.bitcast_convert_type(w & jnp.int32(-65536), jnp.float32)
    return a, b


_TG = 512


def _excl_cumsum(v):
    rows = lax.broadcasted_iota(jnp.int32, v.shape, 0)
    acc = v
    sh = 1
    while sh < v.shape[0]:
        rolled = pltpu.roll(acc, shift=sh, axis=0)
        acc = acc + jnp.where(rows >= sh, rolled, 0)
        sh *= 2
    return acc - v


def _gate_body(x_ref, wg_ref, bg_ref, idx_ref, w_ref, xpk_ref, rank_ref,
               cnt_ref, run_ref):
    i = pl.program_id(0)

    @pl.when(i == 0)
    def _():
        run_ref[...] = jnp.zeros_like(run_ref)

    xv = x_ref[...]
    logits = jnp.dot(xv, wg_ref[...],
                     preferred_element_type=jnp.float32) + bg_ref[...]
    iota = lax.broadcasted_iota(jnp.int32, logits.shape, 1)
    m0 = jnp.max(logits, axis=1, keepdims=True)
    i0 = jnp.min(jnp.where(logits == m0, iota, E), axis=1, keepdims=True)
    masked = jnp.where(iota == i0, -jnp.inf, logits)
    m1 = jnp.max(masked, axis=1, keepdims=True)
    i1 = jnp.min(jnp.where(masked == m1, iota, E), axis=1, keepdims=True)
    e1 = jnp.exp(m1 - m0)
    w0 = 1.0 / (1.0 + e1)
    idx_ref[...] = jnp.concatenate([i0, i1], axis=1)
    w_ref[...] = jnp.concatenate([w0, e1 * w0], axis=1)
    xpk_ref[...] = _pack_bf16(xv[:, :DH], xv[:, DH:])

    run = run_ref[...]
    oh0 = (i0 == iota).astype(jnp.int32)
    oh1 = (i1 == iota).astype(jnp.int32)
    r0 = run + _excl_cumsum(oh0)
    s0 = jnp.sum(oh0, axis=0, keepdims=True)
    r1 = run + s0 + _excl_cumsum(oh1)
    rank_ref[...] = jnp.concatenate(
        [jnp.sum(r0 * oh0, axis=1, keepdims=True),
         jnp.sum(r1 * oh1, axis=1, keepdims=True)], axis=1)
    new_run = run + s0 + jnp.sum(oh1, axis=0, keepdims=True)
    run_ref[...] = new_run
    cnt_ref[...] = new_run


def _gate(x2d, wg, bg):
    return pl.pallas_call(
        _gate_body,
        grid=(N // _TG,),
        in_specs=[
            pl.BlockSpec((_TG, D), lambda i: (i, 0)),
            pl.BlockSpec((D, E), lambda i: (0, 0)),
            pl.BlockSpec((1, E), lambda i: (0, 0)),
        ],
        out_specs=[
            pl.BlockSpec((_TG, K), lambda i: (i, 0)),
            pl.BlockSpec((_TG, K), lambda i: (i, 0)),
            pl.BlockSpec((_TG, DH), lambda i: (i, 0)),
            pl.BlockSpec((_TG, K), lambda i: (i, 0)),
            pl.BlockSpec((1, E), lambda i: (0, 0)),
        ],
        out_shape=[
            jax.ShapeDtypeStruct((N, K), jnp.int32),
            jax.ShapeDtypeStruct((N, K), jnp.float32),
            jax.ShapeDtypeStruct((N, DH), jnp.int32),
            jax.ShapeDtypeStruct((N, K), jnp.int32),
            jax.ShapeDtypeStruct((1, E), jnp.int32),
        ],
        scratch_shapes=[pltpu.VMEM((1, E), jnp.int32)],
    )(x2d, wg, bg.reshape(1, E))


_NW = 32
_CH = 128


def _sc_mesh():
    return plsc.VectorSubcoreMesh(core_axis_name="c", subcore_axis_name="s",
                                  num_cores=2, num_subcores=16)


def _sc_dispatch(xpk, pos_km):
    @functools.partial(
        pl.kernel,
        out_type=jax.ShapeDtypeStruct((A_PAD, DH), jnp.int32),
        mesh=_sc_mesh(),
        scratch_types=[
            pltpu.VMEM((_CH,), jnp.int32),
            pltpu.VMEM((_CH, DH), jnp.int32),
            pltpu.SemaphoreType.DMA,
        ],
    )
    def k(xpk_hbm, pos_hbm, xs_hbm, idx_v, rows_v, sem):
        wid = lax.axis_index("s") * 2 + lax.axis_index("c")
        base = wid * (A // _NW)

        @pl.loop(0, A // _NW, step=_CH)
        def _(off):
            s = base + off
            pltpu.sync_copy(pos_hbm.at[pl.ds(s, _CH)], idx_v)
            t = s - (s // N) * N
            pltpu.sync_copy(xpk_hbm.at[pl.ds(t, _CH)], rows_v)
            pltpu.async_copy(rows_v, xs_hbm.at[idx_v], sem).wait()

    return k(xpk, pos_km)


def _sc_pairs(ys, pos_km):
    @functools.partial(
        pl.kernel,
        out_type=jax.ShapeDtypeStruct((A, DH), jnp.int32),
        mesh=_sc_mesh(),
        scratch_types=[
            pltpu.VMEM((_CH,), jnp.int32),
            pltpu.VMEM((_CH, DH), jnp.int32),
            pltpu.SemaphoreType.DMA,
        ],
    )
    def k(ys_hbm, pos_hbm, out_hbm, idx_v, rows_v, sem):
        wid = lax.axis_index("s") * 2 + lax.axis_index("c")
        base = wid * (A // _NW)

        @pl.loop(0, A // _NW, step=_CH)
        def _(off):
            s = base + off
            pltpu.sync_copy(pos_hbm.at[pl.ds(s, _CH)], idx_v)
            pltpu.async_copy(ys_hbm.at[idx_v], rows_v, sem).wait()
            pltpu.sync_copy(rows_v, out_hbm.at[pl.ds(s, _CH)])

    return k(ys, pos_km)


def _wchanged(te_ref, i):
    return jnp.logical_or(i == 0, te_ref[i] != te_ref[jnp.maximum(i - 1, 0)])


def _ffn1_body(te_ref, used_ref, xs_ref, w1_ref, b1_ref, h_ref, wbf_ref):
    i = pl.program_id(0)

    @pl.when(_wchanged(te_ref, i))
    def _():
        wbf_ref[...] = w1_ref[0].astype(jnp.bfloat16)

    @pl.when(used_ref[i] > 0)
    def _():
        xa, xb = _unpack_bf16(xs_ref[...])
        acc = lax.dot_general(xa.astype(jnp.bfloat16), wbf_ref[:DH],
                              (((1,), (0,)), ((), ())),
                              preferred_element_type=jnp.float32)
        acc += lax.dot_general(xb.astype(jnp.bfloat16), wbf_ref[DH:],
                               (((1,), (0,)), ((), ())),
                               preferred_element_type=jnp.float32)
        acc = acc + b1_ref[0]
        g = 0.5 * acc * (1.0 + lax.erf(acc * _INV_SQRT2))
        h_ref[...] = g.astype(jnp.bfloat16)


def _ffn1(tile_eid, used, xs, w1, b1):
    grid_spec = pltpu.PrefetchScalarGridSpec(
        num_scalar_prefetch=2,
        grid=(NT,),
        in_specs=[
            pl.BlockSpec((TM, DH), lambda i, te, u: (i, 0)),
            pl.BlockSpec((1, D, FF), lambda i, te, u: (te[i], 0, 0)),
            pl.BlockSpec((1, 1, FF), lambda i, te, u: (te[i], 0, 0)),
        ],
        out_specs=pl.BlockSpec((TM, FF), lambda i, te, u: (i, 0)),
        scratch_shapes=[pltpu.VMEM((D, FF), jnp.bfloat16)],
    )
    return pl.pallas_call(
        _ffn1_body,
        grid_spec=grid_spec,
        out_shape=jax.ShapeDtypeStruct((A_PAD, FF), jnp.bfloat16),
    )(tile_eid, used, xs, w1, b1.reshape(E, 1, FF))


def _ffn2_body(te_ref, used_ref, h_ref, w2_ref, b2_ref, ys_ref, wbf_ref):
    i = pl.program_id(0)

    @pl.when(_wchanged(te_ref, i))
    def _():
        wbf_ref[...] = w2_ref[0].astype(jnp.bfloat16)

    @pl.when(used_ref[i] > 0)
    def _():
        y = lax.dot_general(h_ref[...], wbf_ref[...],
                            (((1,), (0,)), ((), ())),
                            preferred_element_type=jnp.float32)
        y = y + b2_ref[0]
        ys_ref[...] = _pack_bf16(y[:, :DH], y[:, DH:])


def _ffn2(tile_eid, used, h, w2, b2):
    grid_spec = pltpu.PrefetchScalarGridSpec(
        num_scalar_prefetch=2,
        grid=(NT,),
        in_specs=[
            pl.BlockSpec((TM, FF), lambda i, te, u: (i, 0)),
            pl.BlockSpec((1, FF, D), lambda i, te, u: (te[i], 0, 0)),
            pl.BlockSpec((1, 1, D), lambda i, te, u: (te[i], 0, 0)),
        ],
        out_specs=pl.BlockSpec((TM, DH), lambda i, te, u: (i, 0)),
        scratch_shapes=[pltpu.VMEM((FF, D), jnp.bfloat16)],
    )
    return pl.pallas_call(
        _ffn2_body,
        grid_spec=grid_spec,
        out_shape=jax.ShapeDtypeStruct((A_PAD, DH), jnp.int32),
    )(tile_eid, used, h, w2, b2.reshape(E, 1, D))


_TP = 512


def _pairsum_body(p0_ref, p1_ref, w_ref, o_ref):
    w = w_ref[...]
    a0, b0 = _unpack_bf16(p0_ref[...])
    a1, b1v = _unpack_bf16(p1_ref[...])
    w0 = w[:, 0:1]
    w1 = w[:, 1:2]
    o_ref[...] = jnp.concatenate(
        [a0 * w0 + a1 * w1, b0 * w0 + b1v * w1], axis=1)


def _pairsum(pairs, wgt):
    return pl.pallas_call(
        _pairsum_body,
        grid=(N // _TP,),
        in_specs=[
            pl.BlockSpec((_TP, DH), lambda i: (i, 0)),
            pl.BlockSpec((_TP, DH), lambda i: (i + N // _TP, 0)),
            pl.BlockSpec((_TP, K), lambda i: (i, 0)),
        ],
        out_specs=pl.BlockSpec((_TP, D), lambda i: (i, 0)),
        out_shape=jax.ShapeDtypeStruct((N, D), jnp.float32),
    )(pairs, pairs, wgt)


def kernel(x, Wg, bg, W1, b1, W2, b2):
    x2d = x.reshape(N, D)
    idx, wgt, xpk, rank, cnt = _gate(x2d, Wg, bg)

    tiles_per = (cnt[0] + TM - 1) // TM
    tile_start = jnp.concatenate(
        [jnp.zeros((1,), jnp.int32), jnp.cumsum(tiles_per)])
    pos = tile_start[idx] * TM + rank
    pos_km = pos.T.reshape(A)
    ti = jnp.arange(NT, dtype=jnp.int32)
    n_used = tile_start[E]
    tile_eid = jnp.minimum(
        jnp.sum((ti[:, None] >= tile_start[None, 1:]).astype(jnp.int32),
                axis=1), E - 1)
    used = (ti < n_used).astype(jnp.int32)

    xs = _sc_dispatch(xpk, pos_km)
    h = _ffn1(tile_eid, used, xs, W1, b1)
    ys = _ffn2(tile_eid, used, h, W2, b2)
    pairs = _sc_pairs(ys, pos_km)
    out2d = _pairsum(pairs, wgt)
    return out2d.reshape(B, T, D)


# speedup vs baseline: 1.0823x; 1.0823x over previous
import functools

import jax
import jax.numpy as jnp
import numpy as np
from jax import lax
from jax.experimental import pallas as pl
from jax.experimental.pallas import tpu as pltpu
from jax.experimental.pallas import tpu_sc as plsc

B, T, D = 2, 2048, 1024
E, K = 8, 2
FF = 4 * D
N = B * T
A = N * K
TM = 256
NT = A // TM + E - 1
A_PAD = NT * TM
DH = D // 2

_INV_SQRT2 = float(1.0 / np.sqrt(2.0))


def _pack_bf16(a, b):
    def rnd(v):
        bits = lax.bitcast_convert_type(v, jnp.int32)
        lsb = lax.shift_right_logical(bits, 16) & 1
        return lax.shift_right_logical(bits + 0x7FFF + lsb, 16)
    return rnd(a) | lax.shift_left(rnd(b), 16)


def _unpack_bf16(w):
    a = lax.bitcast_convert_type(lax.shift_left(w, 16), jnp.float32)
    b = lax.bitcast_convert_type(w & jnp.int32(-65536), jnp.float32)
    return a, b


_TG = 512


def _excl_cumsum(v):
    rows = lax.broadcasted_iota(jnp.int32, v.shape, 0)
    acc = v
    sh = 1
    while sh < v.shape[0]:
        rolled = pltpu.roll(acc, shift=sh, axis=0)
        acc = acc + jnp.where(rows >= sh, rolled, 0)
        sh *= 2
    return acc - v


def _gate_body(x_ref, wg_ref, bg_ref, idx_ref, w_ref, xpk_ref, rank_ref,
               cnt_ref, run_ref):
    i = pl.program_id(0)

    @pl.when(i == 0)
    def _():
        run_ref[...] = jnp.zeros_like(run_ref)

    xv = x_ref[...]
    logits = jnp.dot(xv, wg_ref[...],
                     preferred_element_type=jnp.float32) + bg_ref[...]
    iota = lax.broadcasted_iota(jnp.int32, logits.shape, 1)
    m0 = jnp.max(logits, axis=1, keepdims=True)
    i0 = jnp.min(jnp.where(logits == m0, iota, E), axis=1, keepdims=True)
    masked = jnp.where(iota == i0, -jnp.inf, logits)
    m1 = jnp.max(masked, axis=1, keepdims=True)
    i1 = jnp.min(jnp.where(masked == m1, iota, E), axis=1, keepdims=True)
    e1 = jnp.exp(m1 - m0)
    w0 = 1.0 / (1.0 + e1)
    idx_ref[...] = jnp.concatenate([i0, i1], axis=1)
    w_ref[...] = jnp.concatenate([w0, e1 * w0], axis=1)
    xpk_ref[...] = _pack_bf16(xv[:, :DH], xv[:, DH:])

    run = run_ref[...]
    oh0 = (i0 == iota).astype(jnp.int32)
    oh1 = (i1 == iota).astype(jnp.int32)
    r0 = run + _excl_cumsum(oh0)
    s0 = jnp.sum(oh0, axis=0, keepdims=True)
    r1 = run + s0 + _excl_cumsum(oh1)
    rank_ref[...] = jnp.concatenate(
        [jnp.sum(r0 * oh0, axis=1, keepdims=True),
         jnp.sum(r1 * oh1, axis=1, keepdims=True)], axis=1)
    new_run = run + s0 + jnp.sum(oh1, axis=0, keepdims=True)
    run_ref[...] = new_run
    cnt_ref[...] = new_run


def _gate(x2d, wg, bg):
    return pl.pallas_call(
        _gate_body,
        grid=(N // _TG,),
        in_specs=[
            pl.BlockSpec((_TG, D), lambda i: (i, 0)),
            pl.BlockSpec((D, E), lambda i: (0, 0)),
            pl.BlockSpec((1, E), lambda i: (0, 0)),
        ],
        out_specs=[
            pl.BlockSpec((_TG, K), lambda i: (i, 0)),
            pl.BlockSpec((_TG, K), lambda i: (i, 0)),
            pl.BlockSpec((_TG, DH), lambda i: (i, 0)),
            pl.BlockSpec((_TG, K), lambda i: (i, 0)),
            pl.BlockSpec((1, E), lambda i: (0, 0)),
        ],
        out_shape=[
            jax.ShapeDtypeStruct((N, K), jnp.int32),
            jax.ShapeDtypeStruct((N, K), jnp.float32),
            jax.ShapeDtypeStruct((N, DH), jnp.int32),
            jax.ShapeDtypeStruct((N, K), jnp.int32),
            jax.ShapeDtypeStruct((1, E), jnp.int32),
        ],
        scratch_shapes=[pltpu.VMEM((1, E), jnp.int32)],
    )(x2d, wg, bg.reshape(1, E))


_NW = 32
_CH = 128


def _sc_mesh():
    return plsc.VectorSubcoreMesh(core_axis_name="c", subcore_axis_name="s",
                                  num_cores=2, num_subcores=16)


def _sc_dispatch(xpk, pos_km):
    @functools.partial(
        pl.kernel,
        out_type=jax.ShapeDtypeStruct((A_PAD, DH), jnp.int32),
        mesh=_sc_mesh(),
        scratch_types=[
            pltpu.VMEM((_CH,), jnp.int32),
            pltpu.VMEM((_CH, DH), jnp.int32),
            pltpu.SemaphoreType.DMA,
        ],
    )
    def k(xpk_hbm, pos_hbm, xs_hbm, idx_v, rows_v, sem):
        wid = lax.axis_index("s") * 2 + lax.axis_index("c")
        base = wid * (A // _NW)

        @pl.loop(0, A // _NW, step=_CH)
        def _(off):
            s = base + off
            pltpu.sync_copy(pos_hbm.at[pl.ds(s, _CH)], idx_v)
            t = s - (s // N) * N
            pltpu.sync_copy(xpk_hbm.at[pl.ds(t, _CH)], rows_v)
            pltpu.async_copy(rows_v, xs_hbm.at[idx_v], sem).wait()

    return k(xpk, pos_km)


def _sc_pairs(ys, pos_km):
    @functools.partial(
        pl.kernel,
        out_type=jax.ShapeDtypeStruct((A, DH), jnp.int32),
        mesh=_sc_mesh(),
        scratch_types=[
            pltpu.VMEM((_CH,), jnp.int32),
            pltpu.VMEM((_CH, DH), jnp.int32),
            pltpu.SemaphoreType.DMA,
        ],
    )
    def k(ys_hbm, pos_hbm, out_hbm, idx_v, rows_v, sem):
        wid = lax.axis_index("s") * 2 + lax.axis_index("c")
        base = wid * (A // _NW)

        @pl.loop(0, A // _NW, step=_CH)
        def _(off):
            s = base + off
            pltpu.sync_copy(pos_hbm.at[pl.ds(s, _CH)], idx_v)
            pltpu.async_copy(ys_hbm.at[idx_v], rows_v, sem).wait()
            pltpu.sync_copy(rows_v, out_hbm.at[pl.ds(s, _CH)])

    return k(ys, pos_km)


def _ffn1_body(te_ref, used_ref, xs_ref, w1_ref, b1_ref, h_ref):
    i = pl.program_id(0)

    @pl.when(used_ref[i] > 0)
    def _():
        xa, xb = _unpack_bf16(xs_ref[...])
        w = w1_ref[0]
        acc = lax.dot_general(xa.astype(jnp.bfloat16),
                              w[:DH].astype(jnp.bfloat16),
                              (((1,), (0,)), ((), ())),
                              preferred_element_type=jnp.float32)
        acc += lax.dot_general(xb.astype(jnp.bfloat16),
                               w[DH:].astype(jnp.bfloat16),
                               (((1,), (0,)), ((), ())),
                               preferred_element_type=jnp.float32)
        acc = acc + b1_ref[0]
        g = 0.5 * acc * (1.0 + lax.erf(acc * _INV_SQRT2))
        h_ref[...] = g.astype(jnp.bfloat16)


def _ffn1(tile_eid, used, xs, w1, b1):
    grid_spec = pltpu.PrefetchScalarGridSpec(
        num_scalar_prefetch=2,
        grid=(NT,),
        in_specs=[
            pl.BlockSpec((TM, DH), lambda i, te, u: (i, 0)),
            pl.BlockSpec((1, D, FF), lambda i, te, u: (te[i], 0, 0)),
            pl.BlockSpec((1, 1, FF), lambda i, te, u: (te[i], 0, 0)),
        ],
        out_specs=pl.BlockSpec((TM, FF), lambda i, te, u: (i, 0)),
    )
    return pl.pallas_call(
        _ffn1_body,
        grid_spec=grid_spec,
        out_shape=jax.ShapeDtypeStruct((A_PAD, FF), jnp.bfloat16),
    )(tile_eid, used, xs, w1, b1.reshape(E, 1, FF))


def _ffn2_body(te_ref, used_ref, h_ref, w2_ref, b2_ref, ys_ref):
    i = pl.program_id(0)

    @pl.when(used_ref[i] > 0)
    def _():
        wb = w2_ref[0].astype(jnp.bfloat16)
        y = lax.dot_general(h_ref[...], wb, (((1,), (0,)), ((), ())),
                            preferred_element_type=jnp.float32)
        y = y + b2_ref[0]
        ys_ref[...] = _pack_bf16(y[:, :DH], y[:, DH:])


def _ffn2(tile_eid, used, h, w2, b2):
    grid_spec = pltpu.PrefetchScalarGridSpec(
        num_scalar_prefetch=2,
        grid=(NT,),
        in_specs=[
            pl.BlockSpec((TM, FF), lambda i, te, u: (i, 0)),
            pl.BlockSpec((1, FF, D), lambda i, te, u: (te[i], 0, 0)),
            pl.BlockSpec((1, 1, D), lambda i, te, u: (te[i], 0, 0)),
        ],
        out_specs=pl.BlockSpec((TM, DH), lambda i, te, u: (i, 0)),
    )
    return pl.pallas_call(
        _ffn2_body,
        grid_spec=grid_spec,
        out_shape=jax.ShapeDtypeStruct((A_PAD, DH), jnp.int32),
    )(tile_eid, used, h, w2, b2.reshape(E, 1, D))


_TP = 512


def _pairsum_body(p0_ref, p1_ref, w_ref, o_ref):
    w = w_ref[...]
    a0, b0 = _unpack_bf16(p0_ref[...])
    a1, b1v = _unpack_bf16(p1_ref[...])
    w0 = w[:, 0:1]
    w1 = w[:, 1:2]
    o_ref[...] = jnp.concatenate(
        [a0 * w0 + a1 * w1, b0 * w0 + b1v * w1], axis=1)


def _pairsum(pairs, wgt):
    return pl.pallas_call(
        _pairsum_body,
        grid=(N // _TP,),
        in_specs=[
            pl.BlockSpec((_TP, DH), lambda i: (i, 0)),
            pl.BlockSpec((_TP, DH), lambda i: (i + N // _TP, 0)),
            pl.BlockSpec((_TP, K), lambda i: (i, 0)),
        ],
        out_specs=pl.BlockSpec((_TP, D), lambda i: (i, 0)),
        out_shape=jax.ShapeDtypeStruct((N, D), jnp.float32),
    )(pairs, pairs, wgt)


def kernel(x, Wg, bg, W1, b1, W2, b2):
    x2d = x.reshape(N, D)
    idx, wgt, xpk, rank, cnt = _gate(x2d, Wg, bg)

    tiles_per = (cnt[0] + TM - 1) // TM
    tile_start = jnp.concatenate(
        [jnp.zeros((1,), jnp.int32), jnp.cumsum(tiles_per)])
    pos = tile_start[idx] * TM + rank
    pos_km = pos.T.reshape(A)
    ti = jnp.arange(NT, dtype=jnp.int32)
    n_used = tile_start[E]
    tile_eid = jnp.minimum(
        jnp.sum((ti[:, None] >= tile_start[None, 1:]).astype(jnp.int32),
                axis=1), E - 1)
    used = (ti < n_used).astype(jnp.int32)

    xs = _sc_dispatch(xpk, pos_km)
    h = _ffn1(tile_eid, used, xs, W1, b1)
    ys = _ffn2(tile_eid, used, h, W2, b2)
    pairs = _sc_pairs(ys, pos_km)
    out2d = _pairsum(pairs, wgt)
    return out2d.reshape(B, T, D)


# speedup vs baseline: 1.0933x; 1.0101x over previous
import functools

import jax
import jax.numpy as jnp
import numpy as np
from jax import lax
from jax.experimental import pallas as pl
from jax.experimental.pallas import tpu as pltpu
from jax.experimental.pallas import tpu_sc as plsc

B, T, D = 2, 2048, 1024
E, K = 8, 2
FF = 4 * D
N = B * T
A = N * K
TM = 256
NT = A // TM + E - 1
A_PAD = NT * TM
DH = D // 2

_INV_SQRT2 = float(1.0 / np.sqrt(2.0))


def _pack_bf16(a, b):
    def rnd(v):
        bits = lax.bitcast_convert_type(v, jnp.int32)
        lsb = lax.shift_right_logical(bits, 16) & 1
        return lax.shift_right_logical(bits + 0x7FFF + lsb, 16)
    return rnd(a) | lax.shift_left(rnd(b), 16)


def _unpack_bf16(w):
    a = lax.bitcast_convert_type(lax.shift_left(w, 16), jnp.float32)
    b = lax.bitcast_convert_type(w & jnp.int32(-65536), jnp.float32)
    return a, b


_TG = 1024


def _excl_cumsum(v):
    rows = lax.broadcasted_iota(jnp.int32, v.shape, 0)
    acc = v
    sh = 1
    while sh < v.shape[0]:
        rolled = pltpu.roll(acc, shift=sh, axis=0)
        acc = acc + jnp.where(rows >= sh, rolled, 0)
        sh *= 2
    return acc - v


def _gate_body(x_ref, wg_ref, bg_ref, idx_ref, w_ref, xpk_ref, rank_ref,
               cnt_ref, run_ref):
    i = pl.program_id(0)

    @pl.when(i == 0)
    def _():
        run_ref[...] = jnp.zeros_like(run_ref)

    xv = x_ref[...]
    logits = jnp.dot(xv, wg_ref[...],
                     preferred_element_type=jnp.float32) + bg_ref[...]
    iota = lax.broadcasted_iota(jnp.int32, logits.shape, 1)
    m0 = jnp.max(logits, axis=1, keepdims=True)
    i0 = jnp.min(jnp.where(logits == m0, iota, E), axis=1, keepdims=True)
    masked = jnp.where(iota == i0, -jnp.inf, logits)
    m1 = jnp.max(masked, axis=1, keepdims=True)
    i1 = jnp.min(jnp.where(masked == m1, iota, E), axis=1, keepdims=True)
    e1 = jnp.exp(m1 - m0)
    w0 = 1.0 / (1.0 + e1)
    idx_ref[...] = jnp.concatenate([i0, i1], axis=1)
    w_ref[...] = jnp.concatenate([w0, e1 * w0], axis=1)
    xpk_ref[...] = _pack_bf16(xv[:, :DH], xv[:, DH:])

    run = run_ref[...]
    oh0 = (i0 == iota).astype(jnp.int32)
    oh1 = (i1 == iota).astype(jnp.int32)
    r0 = run + _excl_cumsum(oh0)
    s0 = jnp.sum(oh0, axis=0, keepdims=True)
    r1 = run + s0 + _excl_cumsum(oh1)
    rank_ref[...] = jnp.concatenate(
        [jnp.sum(r0 * oh0, axis=1, keepdims=True),
         jnp.sum(r1 * oh1, axis=1, keepdims=True)], axis=1)
    new_run = run + s0 + jnp.sum(oh1, axis=0, keepdims=True)
    run_ref[...] = new_run
    cnt_ref[...] = new_run


def _gate(x2d, wg, bg):
    return pl.pallas_call(
        _gate_body,
        grid=(N // _TG,),
        in_specs=[
            pl.BlockSpec((_TG, D), lambda i: (i, 0)),
            pl.BlockSpec((D, E), lambda i: (0, 0)),
            pl.BlockSpec((1, E), lambda i: (0, 0)),
        ],
        out_specs=[
            pl.BlockSpec((_TG, K), lambda i: (i, 0)),
            pl.BlockSpec((_TG, K), lambda i: (i, 0)),
            pl.BlockSpec((_TG, DH), lambda i: (i, 0)),
            pl.BlockSpec((_TG, K), lambda i: (i, 0)),
            pl.BlockSpec((1, E), lambda i: (0, 0)),
        ],
        out_shape=[
            jax.ShapeDtypeStruct((N, K), jnp.int32),
            jax.ShapeDtypeStruct((N, K), jnp.float32),
            jax.ShapeDtypeStruct((N, DH), jnp.int32),
            jax.ShapeDtypeStruct((N, K), jnp.int32),
            jax.ShapeDtypeStruct((1, E), jnp.int32),
        ],
        scratch_shapes=[pltpu.VMEM((1, E), jnp.int32)],
    )(x2d, wg, bg.reshape(1, E))


_NW = 32
_CH = 128


def _sc_mesh():
    return plsc.VectorSubcoreMesh(core_axis_name="c", subcore_axis_name="s",
                                  num_cores=2, num_subcores=16)


def _sc_dispatch(xpk, pos_km):
    @functools.partial(
        pl.kernel,
        out_type=jax.ShapeDtypeStruct((A_PAD, DH), jnp.int32),
        mesh=_sc_mesh(),
        scratch_types=[
            pltpu.VMEM((_CH,), jnp.int32),
            pltpu.VMEM((_CH, DH), jnp.int32),
            pltpu.SemaphoreType.DMA,
        ],
    )
    def k(xpk_hbm, pos_hbm, xs_hbm, idx_v, rows_v, sem):
        wid = lax.axis_index("s") * 2 + lax.axis_index("c")
        base = wid * (A // _NW)

        @pl.loop(0, A // _NW, step=_CH)
        def _(off):
            s = base + off
            pltpu.sync_copy(pos_hbm.at[pl.ds(s, _CH)], idx_v)
            t = s - (s // N) * N
            pltpu.sync_copy(xpk_hbm.at[pl.ds(t, _CH)], rows_v)
            pltpu.async_copy(rows_v, xs_hbm.at[idx_v], sem).wait()

    return k(xpk, pos_km)


def _sc_pairs(ys, pos_km):
    @functools.partial(
        pl.kernel,
        out_type=jax.ShapeDtypeStruct((A, DH), jnp.int32),
        mesh=_sc_mesh(),
        scratch_types=[
            pltpu.VMEM((_CH,), jnp.int32),
            pltpu.VMEM((_CH, DH), jnp.int32),
            pltpu.SemaphoreType.DMA,
        ],
    )
    def k(ys_hbm, pos_hbm, out_hbm, idx_v, rows_v, sem):
        wid = lax.axis_index("s") * 2 + lax.axis_index("c")
        base = wid * (A // _NW)

        @pl.loop(0, A // _NW, step=_CH)
        def _(off):
            s = base + off
            pltpu.sync_copy(pos_hbm.at[pl.ds(s, _CH)], idx_v)
            pltpu.async_copy(ys_hbm.at[idx_v], rows_v, sem).wait()
            pltpu.sync_copy(rows_v, out_hbm.at[pl.ds(s, _CH)])

    return k(ys, pos_km)


def _ffn1_body(te_ref, used_ref, xs_ref, w1_ref, b1_ref, h_ref):
    i = pl.program_id(0)

    @pl.when(used_ref[i] > 0)
    def _():
        xa, xb = _unpack_bf16(xs_ref[...])
        w = w1_ref[0]
        acc = lax.dot_general(xa.astype(jnp.bfloat16),
                              w[:DH].astype(jnp.bfloat16),
                              (((1,), (0,)), ((), ())),
                              preferred_element_type=jnp.float32)
        acc += lax.dot_general(xb.astype(jnp.bfloat16),
                               w[DH:].astype(jnp.bfloat16),
                               (((1,), (0,)), ((), ())),
                               preferred_element_type=jnp.float32)
        acc = acc + b1_ref[0]
        g = 0.5 * acc * (1.0 + lax.erf(acc * _INV_SQRT2))
        h_ref[...] = g.astype(jnp.bfloat16)


def _ffn1(tile_eid, used, xs, w1, b1):
    grid_spec = pltpu.PrefetchScalarGridSpec(
        num_scalar_prefetch=2,
        grid=(NT,),
        in_specs=[
            pl.BlockSpec((TM, DH), lambda i, te, u: (i, 0)),
            pl.BlockSpec((1, D, FF), lambda i, te, u: (te[i], 0, 0)),
            pl.BlockSpec((1, 1, FF), lambda i, te, u: (te[i], 0, 0)),
        ],
        out_specs=pl.BlockSpec((TM, FF), lambda i, te, u: (i, 0)),
    )
    return pl.pallas_call(
        _ffn1_body,
        grid_spec=grid_spec,
        out_shape=jax.ShapeDtypeStruct((A_PAD, FF), jnp.bfloat16),
    )(tile_eid, used, xs, w1, b1.reshape(E, 1, FF))


def _ffn2_body(te_ref, used_ref, h_ref, w2_ref, b2_ref, ys_ref):
    i = pl.program_id(0)

    @pl.when(used_ref[i] > 0)
    def _():
        wb = w2_ref[0].astype(jnp.bfloat16)
        y = lax.dot_general(h_ref[...], wb, (((1,), (0,)), ((), ())),
                            preferred_element_type=jnp.float32)
        y = y + b2_ref[0]
        ys_ref[...] = _pack_bf16(y[:, :DH], y[:, DH:])


def _ffn2(tile_eid, used, h, w2, b2):
    grid_spec = pltpu.PrefetchScalarGridSpec(
        num_scalar_prefetch=2,
        grid=(NT,),
        in_specs=[
            pl.BlockSpec((TM, FF), lambda i, te, u: (i, 0)),
            pl.BlockSpec((1, FF, D), lambda i, te, u: (te[i], 0, 0)),
            pl.BlockSpec((1, 1, D), lambda i, te, u: (te[i], 0, 0)),
        ],
        out_specs=pl.BlockSpec((TM, DH), lambda i, te, u: (i, 0)),
    )
    return pl.pallas_call(
        _ffn2_body,
        grid_spec=grid_spec,
        out_shape=jax.ShapeDtypeStruct((A_PAD, DH), jnp.int32),
    )(tile_eid, used, h, w2, b2.reshape(E, 1, D))


_TP = 1024


def _pairsum_body(p0_ref, p1_ref, w_ref, o_ref):
    w = w_ref[...]
    a0, b0 = _unpack_bf16(p0_ref[...])
    a1, b1v = _unpack_bf16(p1_ref[...])
    w0 = w[:, 0:1]
    w1 = w[:, 1:2]
    o_ref[...] = jnp.concatenate(
        [a0 * w0 + a1 * w1, b0 * w0 + b1v * w1], axis=1)


def _pairsum(pairs, wgt):
    return pl.pallas_call(
        _pairsum_body,
        grid=(N // _TP,),
        in_specs=[
            pl.BlockSpec((_TP, DH), lambda i: (i, 0)),
            pl.BlockSpec((_TP, DH), lambda i: (i + N // _TP, 0)),
            pl.BlockSpec((_TP, K), lambda i: (i, 0)),
        ],
        out_specs=pl.BlockSpec((_TP, D), lambda i: (i, 0)),
        out_shape=jax.ShapeDtypeStruct((N, D), jnp.float32),
    )(pairs, pairs, wgt)


def kernel(x, Wg, bg, W1, b1, W2, b2):
    x2d = x.reshape(N, D)
    idx, wgt, xpk, rank, cnt = _gate(x2d, Wg, bg)

    tiles_per = (cnt[0] + TM - 1) // TM
    tile_start = jnp.concatenate(
        [jnp.zeros((1,), jnp.int32), jnp.cumsum(tiles_per)])
    pos = tile_start[idx] * TM + rank
    pos_km = pos.T.reshape(A)
    ti = jnp.arange(NT, dtype=jnp.int32)
    n_used = tile_start[E]
    tile_eid = jnp.minimum(
        jnp.sum((ti[:, None] >= tile_start[None, 1:]).astype(jnp.int32),
                axis=1), E - 1)
    used = (ti < n_used).astype(jnp.int32)

    xs = _sc_dispatch(xpk, pos_km)
    h = _ffn1(tile_eid, used, xs, W1, b1)
    ys = _ffn2(tile_eid, used, h, W2, b2)
    pairs = _sc_pairs(ys, pos_km)
    out2d = _pairsum(pairs, wgt)
    return out2d.reshape(B, T, D)
